# baseline (speedup 1.0000x reference)
import functools

import jax
import jax.numpy as jnp
from jax.experimental import pallas as pl
from jax.experimental.pallas import tpu as pltpu
from jax.experimental.pallas import tpu_sc as plsc

NC = 2
NS = 16
NW = NC * NS
BR = 16
NLANES = 16


def _sc_body(S, D, x_hbm, pos_hbm, o_hbm):
    def block_body(x_vmem, p_vmem, o_vmem):
        for r in range(BR):
            for j in range(D // NLANES):
                slc = (pl.ds(r, 1), pl.ds(j * NLANES, NLANES))
                o_vmem.at[slc][...] = x_vmem.at[slc][...] + p_vmem.at[slc][...]

    R = x_hbm.shape[0]
    spb = S // BR
    pltpu.emit_pipeline(
        block_body,
        grid=(R // BR,),
        in_specs=[
            pl.BlockSpec((BR, D), index_map=lambda i: (i, 0)),
            pl.BlockSpec((BR, D), index_map=lambda i: (jax.lax.rem(i, spb), 0)),
        ],
        out_specs=[pl.BlockSpec((BR, D), index_map=lambda i: (i, 0))],
        core_axis_name=("c", "s"),
        dimension_semantics=(pltpu.PARALLEL,),
    )(x_hbm, pos_hbm, o_hbm)


@functools.lru_cache(maxsize=None)
def _make_sc_call(B, S, D):
    R = B * S
    mesh = plsc.VectorSubcoreMesh(core_axis_name="c", subcore_axis_name="s")
    return pl.kernel(
        functools.partial(_sc_body, S, D),
        out_type=jax.ShapeDtypeStruct((R, D), jnp.float32),
        mesh=mesh,
    )


def kernel(x, pos_table):
    B, S, D = x.shape
    xf = x.reshape(B * S, D)
    out = _make_sc_call(B, S, D)(xf, pos_table)
    return out.reshape(B, S, D)


# speedup vs baseline: 2.1226x; 2.1226x over previous
import functools

import jax
import jax.numpy as jnp
from jax.experimental import pallas as pl
from jax.experimental.pallas import tpu as pltpu
from jax.experimental.pallas import tpu_sc as plsc

NC = 2
NS = 16
NW = NC * NS
BRS = 4
NLANES = 16


def _sc_body(B, S, D, x_hbm, pos_hbm, o_hbm):
    def block_body(x_vmem, p_vmem, o_vmem):
        def load_group(r, j):
            sl = pl.ds(j * NLANES, NLANES)
            p = p_vmem.at[pl.ds(r, 1), sl][...]
            xs = [x_vmem.at[b, pl.ds(r, 1), sl][...] for b in range(B)]
            return (r, j, p, xs)

        def flush_group(g):
            r, j, p, xs = g
            sl = pl.ds(j * NLANES, NLANES)
            for b in range(B):
                o_vmem.at[b, pl.ds(r, 1), sl][...] = xs[b] + p

        prev = None
        for r in range(BRS):
            for j in range(D // NLANES):
                cur = load_group(r, j)
                if prev is not None:
                    flush_group(prev)
                prev = cur
        flush_group(prev)

    pltpu.emit_pipeline(
        block_body,
        grid=(S // BRS,),
        in_specs=[
            pl.BlockSpec((B, BRS, D), index_map=lambda i: (0, i, 0)),
            pl.BlockSpec((BRS, D), index_map=lambda i: (i, 0)),
        ],
        out_specs=[pl.BlockSpec((B, BRS, D), index_map=lambda i: (0, i, 0))],
        core_axis_name=("c", "s"),
        dimension_semantics=(pltpu.PARALLEL,),
    )(x_hbm, pos_hbm, o_hbm)


@functools.lru_cache(maxsize=None)
def _make_sc_call(B, S, D):
    mesh = plsc.VectorSubcoreMesh(core_axis_name="c", subcore_axis_name="s")
    return pl.kernel(
        functools.partial(_sc_body, B, S, D),
        out_type=jax.ShapeDtypeStruct((B, S, D), jnp.float32),
        mesh=mesh,
    )


def kernel(x, pos_table):
    B, S, D = x.shape
    return _make_sc_call(B, S, D)(x, pos_table)


# speedup vs baseline: 2.7382x; 1.2900x over previous
import functools

import jax
import jax.numpy as jnp
from jax import lax
from jax.experimental import pallas as pl
from jax.experimental.pallas import tpu as pltpu
from jax.experimental.pallas import tpu_sc as plsc

NC = 2
NS = 16
NW = NC * NS
CHS = 8
NLANES = 16


def _sc_body(B, S, D, x_hbm, pos_hbm, o_hbm,
             bx0, bx1, bp0, bp1,
             inx0, inx1, inp0, inp1, out0, out1):
    srange = S // NW
    nch = srange // CHS
    wid = lax.axis_index("s") * NC + lax.axis_index("c")
    sbase = wid * srange
    bxs = (bx0, bx1)
    bps = (bp0, bp1)
    inx = (inx0, inx1)
    inp = (inp0, inp1)
    outs = (out0, out1)

    def issue_loads(c, buf):
        soff = sbase + c * CHS
        lx = [pltpu.async_copy(x_hbm.at[pl.ds(b * S + soff, CHS)],
                               bxs[buf].at[b], inx[buf])
              for b in range(B)]
        lp = pltpu.async_copy(pos_hbm.at[pl.ds(soff, CHS)], bps[buf], inp[buf])
        return lx + [lp]

    def issue_store(c, buf):
        soff = sbase + c * CHS
        return [pltpu.async_copy(bxs[buf].at[b],
                                 o_hbm.at[pl.ds(b * S + soff, CHS)], outs[buf])
                for b in range(B)]

    loads = [None] * nch
    stores = [None] * nch
    loads[0] = issue_loads(0, 0)
    for c in range(nch):
        cur = c % 2
        nxt = (c + 1) % 2
        if c + 1 < nch:
            if c - 1 >= 0:
                for d in stores[c - 1]:
                    d.wait()
            loads[c + 1] = issue_loads(c + 1, nxt)
        for d in loads[c]:
            d.wait()

        @pl.loop(0, CHS)
        def _(r):
            X = bxs[cur]
            P = bps[cur]
            for j in range(D // NLANES):
                sl = pl.ds(j * NLANES, NLANES)
                p = P.at[pl.ds(r, 1), sl][...]
                for b in range(B):
                    plsc.addupdate(X.at[b, pl.ds(r, 1), sl], p)

        stores[c] = issue_store(c, cur)
    for c in (nch - 2, nch - 1):
        if 0 <= c < nch:
            for d in stores[c]:
                d.wait()


@functools.lru_cache(maxsize=None)
def _make_sc_call(B, S, D):
    mesh = plsc.VectorSubcoreMesh(core_axis_name="c", subcore_axis_name="s")
    return pl.kernel(
        functools.partial(_sc_body, B, S, D),
        out_type=jax.ShapeDtypeStruct((B * S, D), jnp.float32),
        mesh=mesh,
        scratch_types=[
            pltpu.VMEM((B, CHS, D), jnp.float32),
            pltpu.VMEM((B, CHS, D), jnp.float32),
            pltpu.VMEM((CHS, D), jnp.float32),
            pltpu.VMEM((CHS, D), jnp.float32),
            pltpu.SemaphoreType.DMA,
            pltpu.SemaphoreType.DMA,
            pltpu.SemaphoreType.DMA,
            pltpu.SemaphoreType.DMA,
            pltpu.SemaphoreType.DMA,
            pltpu.SemaphoreType.DMA,
        ],
    )


def kernel(x, pos_table):
    B, S, D = x.shape
    xf = x.reshape(B * S, D)
    out = _make_sc_call(B, S, D)(xf, pos_table)
    return out.reshape(B, S, D)


# speedup vs baseline: 2.7438x; 1.0021x over previous
import functools

import jax
import jax.numpy as jnp
from jax import lax
from jax.experimental import pallas as pl
from jax.experimental.pallas import tpu as pltpu
from jax.experimental.pallas import tpu_sc as plsc

NC = 2
NS = 16
NW = NC * NS
CHS = 8
NLANES = 16


def _sc_body(B, S, D, x_hbm, pos_hbm, o_hbm,
             bx0, bx1, bp0, bp1,
             inx0, inx1, inp0, inp1, out0, out1):
    srange = S // NW
    nch = srange // CHS
    wid = lax.axis_index("s") * NC + lax.axis_index("c")
    sbase = wid * srange
    bxs = (bx0, bx1)
    bps = (bp0, bp1)
    inx = (inx0, inx1)
    inp = (inp0, inp1)
    outs = (out0, out1)

    def issue_loads(c, buf):
        soff = sbase + c * CHS
        lx = [pltpu.async_copy(x_hbm.at[b, pl.ds(soff, CHS)],
                               bxs[buf].at[b], inx[buf])
              for b in range(B)]
        lp = pltpu.async_copy(pos_hbm.at[pl.ds(soff, CHS)], bps[buf], inp[buf])
        return lx + [lp]

    def issue_store(c, buf):
        soff = sbase + c * CHS
        return [pltpu.async_copy(bxs[buf].at[b],
                                 o_hbm.at[b, pl.ds(soff, CHS)], outs[buf])
                for b in range(B)]

    loads = [None] * nch
    stores = [None] * nch
    loads[0] = issue_loads(0, 0)
    for c in range(nch):
        cur = c % 2
        nxt = (c + 1) % 2
        if c + 1 < nch:
            if c - 1 >= 0:
                for d in stores[c - 1]:
                    d.wait()
            loads[c + 1] = issue_loads(c + 1, nxt)
        for d in loads[c]:
            d.wait()

        @pl.loop(0, CHS)
        def _(r):
            X = bxs[cur]
            P = bps[cur]
            for j in range(D // NLANES):
                sl = pl.ds(j * NLANES, NLANES)
                p = P.at[pl.ds(r, 1), sl][...]
                for b in range(B):
                    plsc.addupdate(X.at[b, pl.ds(r, 1), sl], p)

        stores[c] = issue_store(c, cur)
    for c in (nch - 2, nch - 1):
        if 0 <= c < nch:
            for d in stores[c]:
                d.wait()


@functools.lru_cache(maxsize=None)
def _make_sc_call(B, S, D):
    mesh = plsc.VectorSubcoreMesh(core_axis_name="c", subcore_axis_name="s")
    return pl.kernel(
        functools.partial(_sc_body, B, S, D),
        out_type=jax.ShapeDtypeStruct((B, S, D), jnp.float32),
        mesh=mesh,
        scratch_types=[
            pltpu.VMEM((B, CHS, D), jnp.float32),
            pltpu.VMEM((B, CHS, D), jnp.float32),
            pltpu.VMEM((CHS, D), jnp.float32),
            pltpu.VMEM((CHS, D), jnp.float32),
            pltpu.SemaphoreType.DMA,
            pltpu.SemaphoreType.DMA,
            pltpu.SemaphoreType.DMA,
            pltpu.SemaphoreType.DMA,
            pltpu.SemaphoreType.DMA,
            pltpu.SemaphoreType.DMA,
        ],
    )


def kernel(x, pos_table):
    B, S, D = x.shape
    return _make_sc_call(B, S, D)(x, pos_table)
